# baseline (speedup 1.0000x reference)
import functools
import math

import numpy as np
import jax
import jax.numpy as jnp
from jax import lax
from jax.experimental import pallas as pl
from jax.experimental.pallas import tpu as pltpu
from jax.experimental.pallas import tpu_sc as plsc

D_MODEL = 768
LANES = 16
NUM_CORES = 2
NUM_SUBCORES = 16
NUM_WORKERS = NUM_CORES * NUM_SUBCORES
CHUNK_ROWS = 32
NBUF = 4
SCALE = float(np.float32(math.sqrt(D_MODEL)))


def _scale_rows(buf):
    @pl.loop(0, CHUNK_ROWS)
    def _(i):
        row = buf.at[i]
        for j in range(D_MODEL // LANES):
            sl = pl.ds(j * LANES, LANES)
            row[sl] = row[sl] * SCALE


def _emb_body(table_hbm, idx_hbm, out_hbm, idx_v,
              rows0, rows1, rows2, rows3, g0, g1, g2, g3, o0, o1, o2, o3):
    num_chunks = idx_hbm.shape[1]
    wid = lax.axis_index("s") * NUM_CORES + lax.axis_index("c")
    base = wid * (num_chunks * CHUNK_ROWS)
    bufs = (rows0, rows1, rows2, rows3)
    gsems = (g0, g1, g2, g3)
    osems = (o0, o1, o2, o3)
    pltpu.sync_copy(idx_hbm.at[wid], idx_v)

    def gather(c):
        return table_hbm.at[idx_v.at[c]]

    def oslice(c):
        return out_hbm.at[pl.ds(base + c * CHUNK_ROWS, CHUNK_ROWS)]

    out_pending = set()
    for c in range(min(NBUF, num_chunks)):
        pltpu.async_copy(gather(c), bufs[c % NBUF], gsems[c % NBUF])
    for c in range(num_chunks):
        b = c % NBUF
        pltpu.make_async_copy(gather(c), bufs[b], gsems[b]).wait()
        _scale_rows(bufs[b])
        pltpu.async_copy(bufs[b], oslice(c), osems[b])
        out_pending.add(c)
        g = c + 2
        if c >= NBUF - 2 and g < num_chunks:
            prev = g - NBUF
            pltpu.make_async_copy(bufs[prev % NBUF], oslice(prev), osems[prev % NBUF]).wait()
            out_pending.discard(prev)
            pltpu.async_copy(gather(g), bufs[g % NBUF], gsems[g % NBUF])
    for c in sorted(out_pending):
        pltpu.make_async_copy(bufs[c % NBUF], oslice(c), osems[c % NBUF]).wait()


def kernel(x, table):
    batch = x.size
    rows_per_worker = batch // NUM_WORKERS
    num_chunks = rows_per_worker // CHUNK_ROWS
    idx = x.reshape(NUM_WORKERS, num_chunks, CHUNK_ROWS).astype(jnp.int32)

    mesh = plsc.VectorSubcoreMesh(core_axis_name="c", subcore_axis_name="s")
    k = functools.partial(
        pl.kernel,
        out_type=jax.ShapeDtypeStruct((batch, D_MODEL), jnp.float32),
        mesh=mesh,
        scratch_types=(
            [pltpu.VMEM((num_chunks, CHUNK_ROWS), jnp.int32)]
            + [pltpu.VMEM((CHUNK_ROWS, D_MODEL), jnp.float32)] * NBUF
            + [pltpu.SemaphoreType.DMA] * (2 * NBUF)
        ),
    )(_emb_body)
    out = k(table, idx)
    return out.reshape(*x.shape, D_MODEL)
